# Initial kernel scaffold; baseline (speedup 1.0000x reference)
import jax
import jax.numpy as jnp
from jax.experimental import pallas as pl


def kernel(f, edge_index, norm, W):
    raise NotImplementedError("write your pallas kernel here")


# speedup vs baseline: 4.8721x; 4.8721x over previous
import functools

import jax
import jax.numpy as jnp
from jax import lax
from jax.experimental import pallas as pl
from jax.experimental.pallas import tpu as pltpu
from jax.experimental.pallas import tpu_sc as plsc

N_NODES = 10000
N_EDGES = 320000
D = 128
L = 16
NW = 32
NPW = 320
N_PAD = NW * NPW
TRASH = NPW
C = 4000
N_CHUNKS = N_EDGES // C
GROUPS_PER_CHUNK = C // L

_NEG_INF = float("-inf")

_mesh = plsc.VectorSubcoreMesh(
    core_axis_name="c", subcore_axis_name="s", num_cores=2, num_subcores=16
)


@functools.partial(
    pl.kernel,
    out_type=jax.ShapeDtypeStruct((N_PAD, D), jnp.float32),
    mesh=_mesh,
    compiler_params=pltpu.CompilerParams(needs_layout_passes=False),
    scratch_types=[
        pltpu.VMEM((N_NODES,), jnp.float32),
        pltpu.VMEM((NPW + 1, D), jnp.float32),
        pltpu.VMEM((C,), jnp.int32),
        pltpu.VMEM((C,), jnp.int32),
        pltpu.VMEM((C + L,), jnp.int32),
        pltpu.VMEM((L, D), jnp.float32),
        pltpu.VMEM((L,), jnp.int32),
        pltpu.SemaphoreType.DMA,
    ],
)
def _sc_scatter_max(f_hbm, src_hbm, dst_hbm, norm_hbm, s_hbm,
                    norm_v, acc_v, srcc_v, dstc_v, midx_v, rows_v,
                    idx_v, sem):
    wid = lax.axis_index("s") * 2 + lax.axis_index("c")
    lo = wid * NPW

    pltpu.sync_copy(norm_hbm, norm_v)

    def init_body(r, carry):
        for v in range(D // L):
            acc_v[r, pl.ds(v * L, L)] = jnp.full((L,), _NEG_INF, jnp.float32)
        return carry

    lax.fori_loop(0, NPW + 1, init_body, 0)

    lanes = lax.iota(jnp.int32, L)

    def chunk_body(c, carry):
        base = c * C
        pltpu.sync_copy(src_hbm.at[pl.ds(base, C)], srcc_v)
        pltpu.sync_copy(dst_hbm.at[pl.ds(base, C)], dstc_v)

        def scan_body(i, off):
            dv = dstc_v[pl.ds(i * L, L)]
            m = (dv >= lo) & (dv < lo + NPW)
            ids = lanes + i * L
            plsc.store_compressed(midx_v.at[pl.ds(off, L)], ids, mask=m)
            return off + plsc.all_reduce_population_count(m)[0]

        k = lax.fori_loop(0, GROUPS_PER_CHUNK, scan_body, jnp.int32(0))
        n_groups = (k + (L - 1)) // L

        def group_body(g, carry):
            mlane = (g * L + lanes) < k
            idxv = jnp.where(mlane, midx_v[pl.ds(g * L, L)], 0)
            srcs = plsc.load_gather(srcc_v, [idxv])
            dsts = plsc.load_gather(dstc_v, [idxv])
            w = plsc.load_gather(norm_v, [srcs]) * plsc.load_gather(norm_v, [dsts])
            dloc = jnp.where(mlane, dsts - lo, TRASH)
            idx_v[...] = srcs
            pltpu.async_copy(f_hbm.at[idx_v], rows_v, sem).wait()
            for j in range(L):
                dj = dloc[j]
                wv = jnp.full((L,), w[j], jnp.float32)
                for v in range(D // L):
                    sl = pl.ds(v * L, L)
                    acc_v[dj, sl] = jnp.maximum(acc_v[dj, sl], rows_v[j, sl] * wv)
            return carry

        lax.fori_loop(0, n_groups, group_body, 0)
        return carry

    lax.fori_loop(0, N_CHUNKS, chunk_body, 0)

    def fix_body(r, carry):
        for v in range(D // L):
            sl = pl.ds(v * L, L)
            a = acc_v[r, sl]
            acc_v[r, sl] = jnp.where(a == _NEG_INF, 0.0, a)
        return carry

    lax.fori_loop(0, NPW, fix_body, 0)

    pltpu.sync_copy(acc_v.at[pl.ds(0, NPW)], s_hbm.at[pl.ds(lo, NPW)])


def _tc_linear_body(s_ref, w_ref, o_ref):
    o_ref[...] = jnp.maximum(
        lax.dot_general(s_ref[...], w_ref[...], (((1,), (1,)), ((), ())),
                        preferred_element_type=jnp.float32),
        0.0,
    )


def _tc_linear(s_full, W):
    return pl.pallas_call(
        _tc_linear_body,
        out_shape=jax.ShapeDtypeStruct((N_PAD, D), jnp.float32),
    )(s_full, W)


def kernel(f, edge_index, norm, W):
    src = edge_index[0]
    dst = edge_index[1]
    s_full = _sc_scatter_max(f, src, dst, norm.reshape(-1))
    out_full = _tc_linear(s_full, W)
    return (out_full[:N_NODES], s_full[:N_NODES])


# speedup vs baseline: 6.7628x; 1.3881x over previous
import functools

import jax
import jax.numpy as jnp
from jax import lax
from jax.experimental import pallas as pl
from jax.experimental.pallas import tpu as pltpu
from jax.experimental.pallas import tpu_sc as plsc

N_NODES = 10000
N_EDGES = 320000
D = 128
L = 16
NW = 32
NPW = 320
N_PAD = NW * NPW
TRASH = NPW
C = 16000
N_CHUNKS = N_EDGES // C
GROUPS_PER_CHUNK = C // L

_NEG_INF = float("-inf")

_mesh = plsc.VectorSubcoreMesh(
    core_axis_name="c", subcore_axis_name="s", num_cores=2, num_subcores=16
)


@functools.partial(
    pl.kernel,
    out_type=jax.ShapeDtypeStruct((N_PAD, D), jnp.float32),
    mesh=_mesh,
    compiler_params=pltpu.CompilerParams(needs_layout_passes=False),
    scratch_types=[
        pltpu.VMEM((N_NODES,), jnp.float32),
        pltpu.VMEM((NPW + 1, D), jnp.float32),
        pltpu.VMEM((C,), jnp.int32),
        pltpu.VMEM((C,), jnp.int32),
        pltpu.VMEM((C + 3 * L,), jnp.int32),
        pltpu.VMEM((2, L, D), jnp.float32),
        pltpu.VMEM((2, L), jnp.int32),
        pltpu.SemaphoreType.DMA,
        pltpu.SemaphoreType.DMA,
    ],
)
def _sc_scatter_max(f_hbm, src_hbm, dst_hbm, norm_hbm, s_hbm,
                    norm_v, acc_v, srcc_v, dstc_v, midx_v, rows_v,
                    idx_v, sem0, sem1):
    wid = lax.axis_index("s") * 2 + lax.axis_index("c")
    lo = wid * NPW
    sems = (sem0, sem1)

    pltpu.sync_copy(norm_hbm, norm_v)

    def init_body(r, carry):
        for v in range(D // L):
            acc_v[r, pl.ds(v * L, L)] = jnp.full((L,), _NEG_INF, jnp.float32)
        return carry

    lax.fori_loop(0, NPW + 1, init_body, 0)

    lanes = lax.iota(jnp.int32, L)

    def chunk_body(c, carry):
        base = c * C
        pltpu.sync_copy(src_hbm.at[pl.ds(base, C)], srcc_v)
        pltpu.sync_copy(dst_hbm.at[pl.ds(base, C)], dstc_v)

        def scan_body(i, off):
            dv = dstc_v[pl.ds(i * L, L)]
            m = (dv >= lo) & (dv < lo + NPW)
            ids = lanes + i * L
            plsc.store_compressed(midx_v.at[pl.ds(off, L)], ids, mask=m)
            return off + plsc.all_reduce_population_count(m)[0]

        k = lax.fori_loop(0, GROUPS_PER_CHUNK, scan_body, jnp.int32(0))
        n_groups = (k + (L - 1)) // L
        n_pairs = (n_groups + 1) // 2

        def prefetch(g, buf):
            mlane = (g * L + lanes) < k
            idxv = jnp.where(mlane, midx_v[pl.ds(g * L, L)], 0)
            srcs = plsc.load_gather(srcc_v, [idxv])
            dsts = plsc.load_gather(dstc_v, [idxv])
            w = plsc.load_gather(norm_v, [srcs]) * plsc.load_gather(norm_v, [dsts])
            dloc = jnp.where(mlane, dsts - lo, TRASH)
            idx_v[buf, :] = srcs
            pltpu.make_async_copy(f_hbm.at[idx_v.at[buf]],
                                  rows_v.at[buf], sems[buf]).start()
            return w, dloc

        def compute(buf, w, dloc):
            pltpu.make_async_copy(f_hbm.at[idx_v.at[buf]],
                                  rows_v.at[buf], sems[buf]).wait()
            for j in range(L):
                dj = dloc[j]
                wv = jnp.full((L,), w[j], jnp.float32)
                for v in range(D // L):
                    sl = pl.ds(v * L, L)
                    acc_v[dj, sl] = jnp.maximum(acc_v[dj, sl],
                                                rows_v[buf, j, sl] * wv)

        @pl.when(n_groups > 0)
        def _():
            wd0 = prefetch(0, 0)

            def pair_body(p, carry2):
                w0, d0 = carry2
                w1, d1 = prefetch(2 * p + 1, 1)
                compute(0, w0, d0)
                wd0n = prefetch(2 * p + 2, 0)
                compute(1, w1, d1)
                return wd0n

            lax.fori_loop(0, n_pairs, pair_body, wd0)
            pltpu.make_async_copy(f_hbm.at[idx_v.at[0]],
                                  rows_v.at[0], sem0).wait()

        return carry

    lax.fori_loop(0, N_CHUNKS, chunk_body, 0)

    def fix_body(r, carry):
        for v in range(D // L):
            sl = pl.ds(v * L, L)
            a = acc_v[r, sl]
            acc_v[r, sl] = jnp.where(a == _NEG_INF, 0.0, a)
        return carry

    lax.fori_loop(0, NPW, fix_body, 0)

    pltpu.sync_copy(acc_v.at[pl.ds(0, NPW)], s_hbm.at[pl.ds(lo, NPW)])


def _tc_linear_body(s_ref, w_ref, o_ref):
    o_ref[...] = jnp.maximum(
        lax.dot_general(s_ref[...], w_ref[...], (((1,), (1,)), ((), ())),
                        preferred_element_type=jnp.float32),
        0.0,
    )


def _tc_linear(s_full, W):
    return pl.pallas_call(
        _tc_linear_body,
        out_shape=jax.ShapeDtypeStruct((N_PAD, D), jnp.float32),
    )(s_full, W)


def kernel(f, edge_index, norm, W):
    src = edge_index[0]
    dst = edge_index[1]
    s_full = _sc_scatter_max(f, src, dst, norm.reshape(-1))
    out_full = _tc_linear(s_full, W)
    return (out_full[:N_NODES], s_full[:N_NODES])
